# baseline (speedup 1.0000x reference)
import functools

import jax
import jax.numpy as jnp
from jax import lax
from jax.experimental import pallas as pl
from jax.experimental.pallas import tpu as pltpu
from jax.experimental.pallas import tpu_sc as plsc

NC, NS = 2, 16
NW = NC * NS
IDX_W = 128
T = 1024
HT = 512
MT = 8192
D = 128


def _make_sc_gather(n_tokens):
    tpw = n_tokens // NW
    steps = tpw // T
    mesh = plsc.VectorSubcoreMesh(core_axis_name="c", subcore_axis_name="s")

    @functools.partial(
        pl.kernel,
        out_type=jax.ShapeDtypeStruct((n_tokens, D), jnp.float32),
        mesh=mesh,
        compiler_params=pltpu.CompilerParams(use_tc_tiling_on_sc=True),
        scratch_types=[
            pltpu.VMEM((T // IDX_W, IDX_W), jnp.int32),
            pltpu.VMEM((HT, D), jnp.float32),
            pltpu.SemaphoreType.DMA,
        ],
    )
    def sc_gather(idx_hbm, table_hbm, out_hbm, idx_v, rows_v, sem):
        cid = lax.axis_index("c")
        sid = lax.axis_index("s")
        wid = sid * NC + cid
        row_base = wid * (tpw // IDX_W)

        def body(step, carry):
            row_off = row_base + step * (T // IDX_W)
            pltpu.sync_copy(idx_hbm.at[pl.ds(row_off, T // IDX_W)], idx_v)
            for h in range(T // HT):
                cps = [
                    pltpu.async_copy(
                        table_hbm.at[idx_v.at[h * (HT // IDX_W) + j]],
                        rows_v.at[pl.ds(j * IDX_W, IDX_W)], sem)
                    for j in range(HT // IDX_W)
                ]
                for cp in cps:
                    cp.wait()
                pltpu.sync_copy(
                    rows_v, out_hbm.at[pl.ds(row_off * IDX_W + h * HT, HT)])
            return carry

        lax.fori_loop(0, steps, body, 0)

    return sc_gather


def _proj_body(t_ref, w_ref, o_ref):
    o_ref[...] = jnp.dot(t_ref[...], w_ref[...][0:64, :],
                         preferred_element_type=jnp.float32)


def _project_table(url_table, w):
    v, du = url_table.shape
    rb = 4000
    return pl.pallas_call(
        _proj_body,
        grid=(v // rb,),
        in_specs=[
            pl.BlockSpec((rb, du), lambda i: (i, 0)),
            pl.BlockSpec((80, 128), lambda i: (0, 0)),
        ],
        out_specs=pl.BlockSpec((rb, D), lambda i: (i, 0)),
        out_shape=jax.ShapeDtypeStruct((v, D), jnp.float32),
    )(url_table, w)


def _tc_body(g_ref, c_ref, w_ref, b_ref, cat_ref, o_ref):
    w_small = w_ref[...][64:80, :]
    small_proj = jnp.dot(cat_ref[...], w_small,
                         preferred_element_type=jnp.float32)
    code = c_ref[0]
    r = code & 15
    v = (code >> 4) & 127
    q = code >> 11
    ji = lax.broadcasted_iota(jnp.int32, (128, MT), 0)
    mh = (ji == r) | (ji == v + 11) | (ji == q + 111)
    mhf = jnp.where(mh, 1.0, 0.0).astype(jnp.float32)
    small = lax.dot_general(mhf, small_proj, (((0,), (0,)), ((), ())),
                            preferred_element_type=jnp.float32)
    o_ref[...] = jnp.tanh(g_ref[...] + small + b_ref[...])


def _tc_body_prev(prev_ref, g_ref, c_ref, w_ref, b_ref, cat_ref, o_ref):
    del prev_ref
    _tc_body(g_ref, c_ref, w_ref, b_ref, cat_ref, o_ref)


def _tc_call_slab(prev, g_s, c3, w, b2, small_cat, tile_off, n):
    nbs = g_s.shape[0] // MT
    data_specs = [
        pl.BlockSpec((MT, D), lambda i: (i, 0)),
        pl.BlockSpec((1, 1, MT), lambda i: (i, 0, 0)),
        pl.BlockSpec((80, 128), lambda i: (0, 0)),
        pl.BlockSpec((1, 128), lambda i: (0, 0)),
        pl.BlockSpec((128, 16), lambda i: (0, 0)),
    ]
    out_spec = pl.BlockSpec((MT, 128), lambda i: (i + tile_off, 0))
    out_shape = jax.ShapeDtypeStruct((n, 128), jnp.float32)
    if prev is None:
        return pl.pallas_call(
            _tc_body, grid=(nbs,), in_specs=data_specs,
            out_specs=out_spec, out_shape=out_shape,
        )(g_s, c3, w, b2, small_cat)
    return pl.pallas_call(
        _tc_body_prev, grid=(nbs,),
        in_specs=[pl.BlockSpec(memory_space=pl.ANY)] + data_specs,
        out_specs=out_spec, out_shape=out_shape,
        input_output_aliases={0: 0},
    )(prev, g_s, c3, w, b2, small_cat)


def kernel(urls, ranks, vtypes, q_iter, url_table, rank_table, vtype_table,
           qcnt_table, W, b):
    B, L = urls.shape
    n = B * L
    unit = NW * T
    parts = (2, 3, 4, 4, 4, 4, 4)
    url_proj = _project_table(url_table, W)
    idx2 = urls.reshape(n // IDX_W, IDX_W).astype(jnp.int32)
    gather_fns = {u: _make_sc_gather(u * unit) for u in set(parts)}
    g_slabs = []
    row_off = 0
    for u in parts:
        rows = u * unit // IDX_W
        g_slabs.append(
            gather_fns[u](idx2[row_off:row_off + rows], url_proj))
        row_off += rows

    small_cat = jnp.zeros((128, 16), jnp.float32)
    small_cat = small_cat.at[0:11, 0:4].set(rank_table)
    small_cat = small_cat.at[11:111, 4:12].set(vtype_table)
    small_cat = small_cat.at[111:122, 12:16].set(qcnt_table)

    nb = n // MT
    code = (ranks | (vtypes << 4) | (q_iter << 11)).astype(jnp.int32)
    c3 = code.reshape(nb, 1, MT)
    b2 = b.reshape(1, 128)
    out = None
    tile_off = 0
    for s, u in enumerate(parts):
        nbs = u * unit // MT
        sl = slice(tile_off, tile_off + nbs)
        out = _tc_call_slab(out, g_slabs[s], c3[sl],
                            W, b2, small_cat, tile_off, n)
        tile_off += nbs
    return out.reshape(B, L, 128)


# speedup vs baseline: 1.0226x; 1.0226x over previous
import functools

import jax
import jax.numpy as jnp
from jax import lax
from jax.experimental import pallas as pl
from jax.experimental.pallas import tpu as pltpu
from jax.experimental.pallas import tpu_sc as plsc

NC, NS = 2, 16
NW = NC * NS
IDX_W = 128
T = 1024
HT = 512
MT = 4096
D = 128


QT = 256


def _make_sc_gather(n_tokens):
    tpw = n_tokens // NW
    steps = tpw // T
    nq = T // QT
    mesh = plsc.VectorSubcoreMesh(core_axis_name="c", subcore_axis_name="s")

    @functools.partial(
        pl.kernel,
        out_type=jax.ShapeDtypeStruct((n_tokens, D), jnp.float32),
        mesh=mesh,
        compiler_params=pltpu.CompilerParams(use_tc_tiling_on_sc=True),
        scratch_types=[
            pltpu.VMEM((T // IDX_W, IDX_W), jnp.int32),
            pltpu.VMEM((QT, D), jnp.float32),
            pltpu.VMEM((QT, D), jnp.float32),
            pltpu.SemaphoreType.DMA,
            pltpu.SemaphoreType.DMA,
            pltpu.SemaphoreType.DMA,
            pltpu.SemaphoreType.DMA,
        ],
    )
    def sc_gather(idx_hbm, table_hbm, out_hbm, idx_v, rows0, rows1,
                  semg0, semg1, semw0, semw1):
        cid = lax.axis_index("c")
        sid = lax.axis_index("s")
        wid = sid * NC + cid
        row_base = wid * (tpw // IDX_W)
        bufs = (rows0, rows1)
        semg = (semg0, semg1)
        semw = (semw0, semw1)

        def drain_write(par):
            pltpu.make_async_copy(out_hbm.at[pl.ds(0, QT)], bufs[par],
                                  semw[par]).wait()

        def body(step, carry):
            row_off = row_base + step * (T // IDX_W)
            tok0 = row_off * IDX_W
            pltpu.sync_copy(idx_hbm.at[pl.ds(row_off, T // IDX_W)], idx_v)
            gathers = [None, None]
            for q in range(nq):
                par = q & 1
                if q >= 2:
                    drain_write(par)
                else:
                    @pl.when(step > 0)
                    def _():
                        drain_write(par)
                gathers[par] = [
                    pltpu.async_copy(
                        table_hbm.at[idx_v.at[q * (QT // IDX_W) + j]],
                        bufs[par].at[pl.ds(j * IDX_W, IDX_W)], semg[par])
                    for j in range(QT // IDX_W)
                ]
                if q >= 1:
                    prev = 1 - par
                    for cp in gathers[prev]:
                        cp.wait()
                    pltpu.async_copy(
                        bufs[prev],
                        out_hbm.at[pl.ds(tok0 + (q - 1) * QT, QT)],
                        semw[prev])
            last = (nq - 1) & 1
            for cp in gathers[last]:
                cp.wait()
            pltpu.async_copy(bufs[last],
                             out_hbm.at[pl.ds(tok0 + (nq - 1) * QT, QT)],
                             semw[last])
            return carry

        lax.fori_loop(0, steps, body, 0)
        drain_write(0)
        drain_write(1)

    return sc_gather


def _proj_body(t_ref, w_ref, o_ref):
    o_ref[...] = jnp.dot(t_ref[...], w_ref[...][0:64, :],
                         preferred_element_type=jnp.float32)


def _project_table(url_table, w):
    v, du = url_table.shape
    rb = 4000
    return pl.pallas_call(
        _proj_body,
        grid=(v // rb,),
        in_specs=[
            pl.BlockSpec((rb, du), lambda i: (i, 0)),
            pl.BlockSpec((80, 128), lambda i: (0, 0)),
        ],
        out_specs=pl.BlockSpec((rb, D), lambda i: (i, 0)),
        out_shape=jax.ShapeDtypeStruct((v, D), jnp.float32),
    )(url_table, w)


def _tc_body(g_ref, c_ref, w_ref, b_ref, cat_ref, o_ref):
    w_small = w_ref[...][64:80, :]
    small_proj = jnp.dot(cat_ref[...], w_small,
                         preferred_element_type=jnp.float32)
    code = c_ref[0]
    r = code & 15
    v = (code >> 4) & 127
    q = code >> 11
    ji = lax.broadcasted_iota(jnp.int32, (128, MT), 0)
    mh = (ji == r) | (ji == v + 11) | (ji == q + 111)
    mhf = jnp.where(mh, 1.0, 0.0).astype(jnp.float32)
    small = lax.dot_general(mhf, small_proj, (((0,), (0,)), ((), ())),
                            preferred_element_type=jnp.float32)
    o_ref[...] = jnp.tanh(g_ref[...] + small + b_ref[...])


def _tc_body_prev(prev_ref, g_ref, c_ref, w_ref, b_ref, cat_ref, o_ref):
    del prev_ref
    _tc_body(g_ref, c_ref, w_ref, b_ref, cat_ref, o_ref)


def _tc_call_slab(prev, g_s, c3, w, b2, small_cat, tile_off, n):
    nbs = g_s.shape[0] // MT
    data_specs = [
        pl.BlockSpec((MT, D), lambda i: (i, 0)),
        pl.BlockSpec((1, 1, MT), lambda i: (i, 0, 0)),
        pl.BlockSpec((80, 128), lambda i: (0, 0)),
        pl.BlockSpec((1, 128), lambda i: (0, 0)),
        pl.BlockSpec((128, 16), lambda i: (0, 0)),
    ]
    out_spec = pl.BlockSpec((MT, 128), lambda i: (i + tile_off, 0))
    out_shape = jax.ShapeDtypeStruct((n, 128), jnp.float32)
    if prev is None:
        return pl.pallas_call(
            _tc_body, grid=(nbs,), in_specs=data_specs,
            out_specs=out_spec, out_shape=out_shape,
        )(g_s, c3, w, b2, small_cat)
    return pl.pallas_call(
        _tc_body_prev, grid=(nbs,),
        in_specs=[pl.BlockSpec(memory_space=pl.ANY)] + data_specs,
        out_specs=out_spec, out_shape=out_shape,
        input_output_aliases={0: 0},
    )(prev, g_s, c3, w, b2, small_cat)


def kernel(urls, ranks, vtypes, q_iter, url_table, rank_table, vtype_table,
           qcnt_table, W, b):
    B, L = urls.shape
    n = B * L
    unit = NW * T
    parts = (2, 3, 4, 4, 4, 4, 4)
    url_proj = _project_table(url_table, W)
    idx2 = urls.reshape(n // IDX_W, IDX_W).astype(jnp.int32)
    gather_fns = {u: _make_sc_gather(u * unit) for u in set(parts)}
    g_slabs = []
    row_off = 0
    for u in parts:
        rows = u * unit // IDX_W
        g_slabs.append(
            gather_fns[u](idx2[row_off:row_off + rows], url_proj))
        row_off += rows

    small_cat = jnp.zeros((128, 16), jnp.float32)
    small_cat = small_cat.at[0:11, 0:4].set(rank_table)
    small_cat = small_cat.at[11:111, 4:12].set(vtype_table)
    small_cat = small_cat.at[111:122, 12:16].set(qcnt_table)

    nb = n // MT
    code = (ranks | (vtypes << 4) | (q_iter << 11)).astype(jnp.int32)
    c3 = code.reshape(nb, 1, MT)
    b2 = b.reshape(1, 128)
    out = None
    tile_off = 0
    for s, u in enumerate(parts):
        nbs = u * unit // MT
        sl = slice(tile_off, tile_off + nbs)
        out = _tc_call_slab(out, g_slabs[s], c3[sl],
                            W, b2, small_cat, tile_off, n)
        tile_off += nbs
    return out.reshape(B, L, 128)
